# baseline (speedup 1.0000x reference)
import functools

import jax
import jax.numpy as jnp
from jax.experimental import pallas as pl

_PAD = 16


def _attn_block(x_ref, wt_ref, wo_ref, pool_ref, gamma_ref, out_ref):
    fd = wt_ref.shape[0]
    xb = x_ref[0]
    theta = jax.lax.dot_general(
        wt_ref[:], xb, (((1,), (0,)), ((), ())),
        preferred_element_type=jnp.float32)
    theta_bf = theta.astype(jnp.bfloat16)
    logits = jax.lax.dot_general(
        pool_ref[0:fd], theta_bf, (((0,), (0,)), ((), ())),
        preferred_element_type=jnp.float32)
    m = jnp.max(logits, axis=0, keepdims=True)
    e = jnp.exp(logits - m).astype(jnp.bfloat16)
    agg_aug = jax.lax.dot_general(
        pool_ref[:], e, (((1,), (0,)), ((), ())),
        preferred_element_type=jnp.float32)
    s = agg_aug[fd:fd + 1]
    agg = agg_aug[0:fd] / s
    o = jax.lax.dot_general(
        wo_ref[:], agg, (((1,), (0,)), ((), ())),
        preferred_element_type=jnp.float32)
    out_ref[0] = gamma_ref[0, 0] * o + xb


@functools.partial(jax.jit, static_argnames=("n_blk",))
def _run(x, W_theta, W_o, concept_pool, gamma, n_blk=256):
    B, C, H, W = x.shape
    fd, P = concept_pool.shape
    n = H * W
    xr = x.reshape(B, C, n)
    pool_aug = jnp.concatenate(
        [concept_pool.astype(jnp.bfloat16),
         jnp.ones((_PAD, P), jnp.bfloat16)], axis=0)
    grid = (B, n // n_blk)
    out = pl.pallas_call(
        _attn_block,
        grid=grid,
        in_specs=[
            pl.BlockSpec((1, C, n_blk), lambda b, j: (b, 0, j)),
            pl.BlockSpec((fd, C), lambda b, j: (0, 0)),
            pl.BlockSpec((C, fd), lambda b, j: (0, 0)),
            pl.BlockSpec((fd + _PAD, P), lambda b, j: (0, 0)),
            pl.BlockSpec((1, 1), lambda b, j: (0, 0)),
        ],
        out_specs=pl.BlockSpec((1, C, n_blk), lambda b, j: (b, 0, j)),
        out_shape=jax.ShapeDtypeStruct((B, C, n), jnp.float32),
    )(xr, W_theta, W_o, pool_aug, jnp.reshape(gamma, (1, 1)))
    return out.reshape(B, C, H, W)


def kernel(x, W_theta, W_o, concept_pool, gamma):
    return _run(x, W_theta, W_o, concept_pool, gamma)


# speedup vs baseline: 1.5807x; 1.5807x over previous
import functools

import jax
import jax.numpy as jnp
from jax.experimental import pallas as pl

_PAD = 16


def _attn_block(x_ref, wt_ref, wo_ref, pool_ref, gamma_ref, out_ref):
    fd = wt_ref.shape[0]
    nb = x_ref.shape[2]
    xb = x_ref[0]
    theta = jax.lax.dot_general(
        wt_ref[:], xb, (((1,), (0,)), ((), ())),
        preferred_element_type=jnp.float32)
    b = jnp.sum(jnp.maximum(theta, 0.0), axis=0, keepdims=True)
    shift = jnp.broadcast_to(-b / _PAD, (_PAD, nb))
    theta_aug = jnp.concatenate(
        [theta, shift], axis=0).astype(jnp.bfloat16)
    logits = jax.lax.dot_general(
        pool_ref[:], theta_aug, (((0,), (0,)), ((), ())),
        preferred_element_type=jnp.float32)
    e = jnp.exp(logits).astype(jnp.bfloat16)
    agg_aug = jax.lax.dot_general(
        pool_ref[:], e, (((1,), (0,)), ((), ())),
        preferred_element_type=jnp.float32)
    s = agg_aug[fd:fd + 1]
    agg = agg_aug[0:fd] / s
    o = jax.lax.dot_general(
        wo_ref[:], agg, (((1,), (0,)), ((), ())),
        preferred_element_type=jnp.float32)
    out_ref[0] = gamma_ref[0, 0] * o + xb


@functools.partial(jax.jit, static_argnames=("n_blk",))
def _run(x, W_theta, W_o, concept_pool, gamma, n_blk=256):
    B, C, H, W = x.shape
    fd, P = concept_pool.shape
    n = H * W
    xr = x.reshape(B, C, n)
    pool_aug = jnp.concatenate(
        [concept_pool.astype(jnp.bfloat16),
         jnp.ones((_PAD, P), jnp.bfloat16)], axis=0)
    grid = (B, n // n_blk)
    out = pl.pallas_call(
        _attn_block,
        grid=grid,
        in_specs=[
            pl.BlockSpec((1, C, n_blk), lambda b, j: (b, 0, j)),
            pl.BlockSpec((fd, C), lambda b, j: (0, 0)),
            pl.BlockSpec((C, fd), lambda b, j: (0, 0)),
            pl.BlockSpec((fd + _PAD, P), lambda b, j: (0, 0)),
            pl.BlockSpec((1, 1), lambda b, j: (0, 0)),
        ],
        out_specs=pl.BlockSpec((1, C, n_blk), lambda b, j: (b, 0, j)),
        out_shape=jax.ShapeDtypeStruct((B, C, n), jnp.float32),
    )(xr, W_theta, W_o, pool_aug, jnp.reshape(gamma, (1, 1)))
    return out.reshape(B, C, H, W)


def kernel(x, W_theta, W_o, concept_pool, gamma):
    return _run(x, W_theta, W_o, concept_pool, gamma)


# speedup vs baseline: 1.7998x; 1.1386x over previous
import functools

import jax
import jax.numpy as jnp
from jax.experimental import pallas as pl

_PAD = 16


def _attn_block(x_ref, wt_ref, wo_ref, pool_ref, pool_t_ref, gamma_ref, out_ref):
    fd = wt_ref.shape[0]
    nb = x_ref.shape[2]
    xb = x_ref[0]
    theta = jax.lax.dot_general(
        wt_ref[:], xb, (((1,), (0,)), ((), ())),
        preferred_element_type=jnp.float32)
    b = jnp.sum(jnp.maximum(theta, 0.0), axis=0, keepdims=True)
    shift = jnp.broadcast_to(-b / _PAD, (_PAD, nb))
    theta_aug = jnp.concatenate(
        [theta, shift], axis=0).astype(jnp.bfloat16)
    logits = jax.lax.dot_general(
        pool_t_ref[:], theta_aug, (((1,), (0,)), ((), ())),
        preferred_element_type=jnp.float32)
    e = jnp.exp2(logits).astype(jnp.bfloat16)
    agg_aug = jax.lax.dot_general(
        pool_ref[:], e, (((1,), (0,)), ((), ())),
        preferred_element_type=jnp.float32)
    s = agg_aug[fd:fd + 1]
    agg = agg_aug[0:fd] / s
    o = jax.lax.dot_general(
        wo_ref[:], agg, (((1,), (0,)), ((), ())),
        preferred_element_type=jnp.float32)
    out_ref[0] = gamma_ref[0, 0] * o + xb


@functools.partial(jax.jit, static_argnames=("n_blk",))
def _run(x, W_theta, W_o, concept_pool, gamma, n_blk=1024):
    B, C, H, W = x.shape
    fd, P = concept_pool.shape
    n = H * W
    xr = x.reshape(B, C, n)
    W_theta = W_theta * jnp.float32(1.4426950408889634)
    pool_aug = jnp.concatenate(
        [concept_pool.astype(jnp.bfloat16),
         jnp.ones((_PAD, P), jnp.bfloat16)], axis=0)
    pool_t = pool_aug.T
    grid = (B, n // n_blk)
    out = pl.pallas_call(
        _attn_block,
        grid=grid,
        in_specs=[
            pl.BlockSpec((1, C, n_blk), lambda b, j: (b, 0, j)),
            pl.BlockSpec((fd, C), lambda b, j: (0, 0)),
            pl.BlockSpec((C, fd), lambda b, j: (0, 0)),
            pl.BlockSpec((fd + _PAD, P), lambda b, j: (0, 0)),
            pl.BlockSpec((P, fd + _PAD), lambda b, j: (0, 0)),
            pl.BlockSpec((1, 1), lambda b, j: (0, 0)),
        ],
        out_specs=pl.BlockSpec((1, C, n_blk), lambda b, j: (b, 0, j)),
        out_shape=jax.ShapeDtypeStruct((B, C, n), jnp.float32),
    )(xr, W_theta, W_o, pool_aug, pool_t, jnp.reshape(gamma, (1, 1)))
    return out.reshape(B, C, H, W)


def kernel(x, W_theta, W_o, concept_pool, gamma):
    return _run(x, W_theta, W_o, concept_pool, gamma)


# speedup vs baseline: 1.9561x; 1.0868x over previous
import functools

import jax
import jax.numpy as jnp
from jax.experimental import pallas as pl
from jax.experimental.pallas import tpu as pltpu

_PAD = 16
_LOG2E = 1.4426950408889634


def _attn_block(x_ref, wt_ref, wo_ref, pool_ref, gamma_ref, out_ref, pa_ref):
    fd = pool_ref.shape[0]
    nb = x_ref.shape[2]

    @pl.when((pl.program_id(0) == 0) & (pl.program_id(1) == 0))
    def _init():
        pa_ref[0:fd, :] = pool_ref[:].astype(jnp.bfloat16)
        pa_ref[fd:, :] = jnp.ones((_PAD, pool_ref.shape[1]), jnp.bfloat16)

    xb = x_ref[0]
    theta = jax.lax.dot_general(
        wt_ref[:] * jnp.float32(_LOG2E), xb, (((1,), (0,)), ((), ())),
        preferred_element_type=jnp.float32)
    b = jnp.sum(jnp.maximum(theta, 0.0), axis=0, keepdims=True)
    shift = jnp.broadcast_to(-b / _PAD, (_PAD, nb))
    theta_aug = jnp.concatenate(
        [theta, shift], axis=0).astype(jnp.bfloat16)
    logits = jax.lax.dot_general(
        pa_ref[:], theta_aug, (((0,), (0,)), ((), ())),
        preferred_element_type=jnp.float32)
    e = jnp.exp2(logits).astype(jnp.bfloat16)
    agg_aug = jax.lax.dot_general(
        pa_ref[:], e, (((1,), (0,)), ((), ())),
        preferred_element_type=jnp.float32)
    s = agg_aug[fd:fd + 1]
    agg = agg_aug[0:fd] / s
    o = jax.lax.dot_general(
        wo_ref[:], agg, (((1,), (0,)), ((), ())),
        preferred_element_type=jnp.float32)
    out_ref[0] = gamma_ref[0, 0] * o + xb


@functools.partial(jax.jit, static_argnames=("n_blk",))
def _run(x, W_theta, W_o, concept_pool, gamma, n_blk=1024):
    B, C, H, W = x.shape
    fd, P = concept_pool.shape
    n = H * W
    xr = x.reshape(B, C, n)
    grid = (B, n // n_blk)
    out = pl.pallas_call(
        _attn_block,
        grid=grid,
        in_specs=[
            pl.BlockSpec((1, C, n_blk), lambda b, j: (b, 0, j)),
            pl.BlockSpec((fd, C), lambda b, j: (0, 0)),
            pl.BlockSpec((C, fd), lambda b, j: (0, 0)),
            pl.BlockSpec((fd, P), lambda b, j: (0, 0)),
            pl.BlockSpec((1, 1), lambda b, j: (0, 0)),
        ],
        out_specs=pl.BlockSpec((1, C, n_blk), lambda b, j: (b, 0, j)),
        out_shape=jax.ShapeDtypeStruct((B, C, n), jnp.float32),
        scratch_shapes=[pltpu.VMEM((fd + _PAD, P), jnp.bfloat16)],
    )(xr, W_theta, W_o, concept_pool, jnp.reshape(gamma, (1, 1)))
    return out.reshape(B, C, H, W)


def kernel(x, W_theta, W_o, concept_pool, gamma):
    return _run(x, W_theta, W_o, concept_pool, gamma)


# speedup vs baseline: 1.9596x; 1.0018x over previous
import functools

import jax
import jax.numpy as jnp
from jax.experimental import pallas as pl
from jax.experimental.pallas import tpu as pltpu

_PAD = 16
_LOG2E = 1.4426950408889634


def _attn_block(x_ref, wt_ref, wo_ref, pool_ref, gamma_ref, out_ref, pa_ref):
    fd = pool_ref.shape[0]
    nb = x_ref.shape[2]

    @pl.when(pl.program_id(1) == 0)
    def _init():
        pa_ref[0:fd, :] = pool_ref[:].astype(jnp.bfloat16)
        pa_ref[fd:, :] = jnp.ones((_PAD, pool_ref.shape[1]), jnp.bfloat16)

    xb = x_ref[0]
    theta = jax.lax.dot_general(
        wt_ref[:] * jnp.float32(_LOG2E), xb, (((1,), (0,)), ((), ())),
        preferred_element_type=jnp.float32)
    b = jnp.sum(jnp.maximum(theta, 0.0), axis=0, keepdims=True)
    shift = jnp.broadcast_to(-b / _PAD, (_PAD, nb))
    theta_aug = jnp.concatenate(
        [theta, shift], axis=0).astype(jnp.bfloat16)
    logits = jax.lax.dot_general(
        pa_ref[:], theta_aug, (((0,), (0,)), ((), ())),
        preferred_element_type=jnp.float32)
    e = jnp.exp2(logits).astype(jnp.bfloat16)
    agg_aug = jax.lax.dot_general(
        pa_ref[:], e, (((1,), (0,)), ((), ())),
        preferred_element_type=jnp.float32)
    s = agg_aug[fd:fd + 1]
    agg = agg_aug[0:fd] / s
    o = jax.lax.dot_general(
        wo_ref[:], agg, (((1,), (0,)), ((), ())),
        preferred_element_type=jnp.float32)
    out_ref[0] = gamma_ref[0, 0] * o + xb


@functools.partial(jax.jit, static_argnames=("n_blk",))
def _run(x, W_theta, W_o, concept_pool, gamma, n_blk=1024):
    B, C, H, W = x.shape
    fd, P = concept_pool.shape
    n = H * W
    xr = x.reshape(B, C, n)
    grid = (B, n // n_blk)
    out = pl.pallas_call(
        _attn_block,
        grid=grid,
        in_specs=[
            pl.BlockSpec((1, C, n_blk), lambda b, j: (b, 0, j)),
            pl.BlockSpec((fd, C), lambda b, j: (0, 0)),
            pl.BlockSpec((C, fd), lambda b, j: (0, 0)),
            pl.BlockSpec((fd, P), lambda b, j: (0, 0)),
            pl.BlockSpec((1, 1), lambda b, j: (0, 0)),
        ],
        out_specs=pl.BlockSpec((1, C, n_blk), lambda b, j: (b, 0, j)),
        out_shape=jax.ShapeDtypeStruct((B, C, n), jnp.float32),
        scratch_shapes=[pltpu.VMEM((fd + _PAD, P), jnp.bfloat16)],
        compiler_params=pltpu.CompilerParams(
            dimension_semantics=("parallel", "arbitrary")),
    )(xr, W_theta, W_o, concept_pool, jnp.reshape(gamma, (1, 1)))
    return out.reshape(B, C, H, W)


def kernel(x, W_theta, W_o, concept_pool, gamma):
    return _run(x, W_theta, W_o, concept_pool, gamma)
